# Initial kernel scaffold; baseline (speedup 1.0000x reference)
import jax
import jax.numpy as jnp
from jax.experimental import pallas as pl


def kernel(h, phrase_mask, phrase_token_idx, W_kv, W_z, B_pos):
    raise NotImplementedError("write your pallas kernel here")


# speedup vs baseline: 9.2678x; 9.2678x over previous
import functools

import jax
import jax.numpy as jnp
from jax import lax
from jax.experimental import pallas as pl
from jax.experimental.pallas import tpu as pltpu
from jax.experimental.pallas import tpu_sc as plsc

B, T, D = 4, 8192, 768
P, LMAX, C = 4096, 8, 64
BP = B * P
CAT = 2 * C
NC, NS = 2, 16
NW = NC * NS
PPW = BP // NW
CHUNK = 16
NCHUNK = PPW // CHUNK

_NEG = -1e30


def _mm_body(x_ref, w_ref, o_ref):
    o_ref[...] = jnp.dot(x_ref[...], w_ref[...],
                         preferred_element_type=jnp.float32)


def _project(x, w_cat):
    bm = 1024
    return pl.pallas_call(
        _mm_body,
        grid=(x.shape[0] // bm,),
        in_specs=[pl.BlockSpec((bm, D), lambda i: (i, 0)),
                  pl.BlockSpec((D, CAT), lambda i: (0, 0))],
        out_specs=pl.BlockSpec((bm, CAT), lambda i: (i, 0)),
        out_shape=jax.ShapeDtypeStruct((x.shape[0], CAT), jnp.float32),
    )(x, w_cat)


_mesh = plsc.VectorSubcoreMesh(core_axis_name="c", subcore_axis_name="s")


@functools.partial(
    pl.kernel,
    mesh=_mesh,
    out_type=jax.ShapeDtypeStruct((BP, C), jnp.float32),
    scratch_types=[
        pltpu.VMEM((CHUNK * LMAX,), jnp.int32),
        pltpu.VMEM((CHUNK * LMAX, 16), jnp.float32),
        pltpu.VMEM((LMAX * C,), jnp.float32),
        pltpu.VMEM((CHUNK * LMAX, CAT), jnp.float32),
        pltpu.VMEM((CHUNK, C), jnp.float32),
        pltpu.SemaphoreType.DMA,
    ],
)
def _sc_pool(cat_hbm, idx_hbm, mb_hbm, bpos_hbm, out_hbm,
             idx_v, mb_v, bpos_v, rows_v, out_v, sem):
    wid = lax.axis_index("s") * NC + lax.axis_index("c")
    tok_off = (wid // (NW // B)) * T

    pltpu.sync_copy(bpos_hbm, bpos_v)
    bpos = [[bpos_v[pl.ds(l * C + 16 * j, 16)] for j in range(C // 16)]
            for l in range(LMAX)]

    def chunk_body(ci, carry):
        start = wid * PPW + ci * CHUNK
        pltpu.sync_copy(idx_hbm.at[pl.ds(start * LMAX, CHUNK * LMAX)], idx_v)
        pltpu.sync_copy(mb_hbm.at[pl.ds(start * LMAX, CHUNK * LMAX)], mb_v)
        for i in range(CHUNK * LMAX // 16):
            sl = pl.ds(16 * i, 16)
            idx_v[sl] = idx_v[sl] + tok_off
        pltpu.async_copy(cat_hbm.at[idx_v], rows_v, sem).wait()

        def phrase_body(p, c2):
            base = p * LMAX
            mb = [mb_v[base + l, pl.ds(0, 16)] for l in range(LMAX)]
            for j in range(C // 16):
                zb = [rows_v[base + l, pl.ds(C + 16 * j, 16)]
                      + bpos[l][j] + mb[l] for l in range(LMAX)]
                m = zb[0]
                for l in range(1, LMAX):
                    m = jnp.maximum(m, zb[l])
                e = [jnp.exp(zb[l] - m) for l in range(LMAX)]
                s = e[0]
                for l in range(1, LMAX):
                    s = s + e[l]
                acc = e[0] * rows_v[base, pl.ds(16 * j, 16)]
                for l in range(1, LMAX):
                    acc = acc + e[l] * rows_v[base + l, pl.ds(16 * j, 16)]
                out_v[p, pl.ds(16 * j, 16)] = acc / s
            return c2

        lax.fori_loop(0, CHUNK, phrase_body, 0)
        pltpu.sync_copy(out_v, out_hbm.at[pl.ds(start, CHUNK)])
        return carry

    lax.fori_loop(0, NCHUNK, chunk_body, 0)


def kernel(h, phrase_mask, phrase_token_idx, W_kv, W_z, B_pos):
    x = h.reshape(B * T, D)
    w_cat = jnp.concatenate([W_kv, W_z], axis=1)
    cat = _project(x, w_cat)
    idx_flat = phrase_token_idx.astype(jnp.int32).reshape(-1)
    mb = jnp.where(phrase_mask, 0.0, _NEG).astype(jnp.float32).reshape(-1)
    mb_exp = jnp.broadcast_to(mb[:, None], (BP * LMAX, 16))
    out = _sc_pool(cat, idx_flat, mb_exp,
                   B_pos.astype(jnp.float32).reshape(-1))
    return out.reshape(B, P, C)


# speedup vs baseline: 16.7559x; 1.8080x over previous
import functools

import jax
import jax.numpy as jnp
from jax import lax
from jax.experimental import pallas as pl
from jax.experimental.pallas import tpu as pltpu
from jax.experimental.pallas import tpu_sc as plsc

B, T, D = 4, 8192, 768
P, LMAX, C = 4096, 8, 64
BP = B * P
CAT = 2 * C
NC, NS = 2, 16
NW = NC * NS
PPW = BP // NW
CHUNK = 16
NCHUNK = PPW // CHUNK
RPC = CHUNK * LMAX

_NEG = -1e30


def _mm_body(x_ref, w_ref, o_ref):
    o_ref[...] = jnp.dot(x_ref[...], w_ref[...],
                         preferred_element_type=jnp.float32)


def _project(x, w_cat):
    bm = 1024
    return pl.pallas_call(
        _mm_body,
        grid=(x.shape[0] // bm,),
        in_specs=[pl.BlockSpec((bm, D), lambda i: (i, 0)),
                  pl.BlockSpec((D, CAT), lambda i: (0, 0))],
        out_specs=pl.BlockSpec((bm, CAT), lambda i: (i, 0)),
        out_shape=jax.ShapeDtypeStruct((x.shape[0], CAT), jnp.float32),
    )(x, w_cat)


_mesh = plsc.VectorSubcoreMesh(core_axis_name="c", subcore_axis_name="s")


@functools.partial(
    pl.kernel,
    mesh=_mesh,
    out_type=jax.ShapeDtypeStruct((BP, C), jnp.float32),
    scratch_types=[
        pltpu.VMEM((PPW * LMAX,), jnp.int32),
        pltpu.VMEM((PPW, LMAX * 16), jnp.float32),
        pltpu.VMEM((LMAX * C,), jnp.float32),
        pltpu.VMEM((2, RPC, CAT), jnp.float32),
        pltpu.VMEM((2, CHUNK, C), jnp.float32),
        pltpu.SemaphoreType.DMA,
        pltpu.SemaphoreType.DMA,
        pltpu.SemaphoreType.DMA,
        pltpu.SemaphoreType.DMA,
    ],
)
def _sc_pool(cat_hbm, idx_hbm, mb_hbm, bpos_hbm, out_hbm,
             idx_v, mb_v, bpos_v, rows_v, out_v, gsem0, gsem1, osem0, osem1):
    wid = lax.axis_index("s") * NC + lax.axis_index("c")
    tok_off = (wid // (NW // B)) * T
    start_w = wid * PPW
    gsem = (gsem0, gsem1)
    osem = (osem0, osem1)

    pltpu.sync_copy(bpos_hbm, bpos_v)
    pltpu.sync_copy(idx_hbm.at[pl.ds(start_w * LMAX, PPW * LMAX)], idx_v)
    pltpu.sync_copy(mb_hbm.at[pl.ds(start_w, PPW)], mb_v)

    def add_off(i, carry):
        sl = pl.ds(16 * i, 16)
        idx_v[sl] = idx_v[sl] + tok_off
        return carry

    lax.fori_loop(0, PPW * LMAX // 16, add_off, 0)

    bpos = [[bpos_v[pl.ds(l * C + 16 * j, 16)] for j in range(C // 16)]
            for l in range(LMAX)]

    def _gather(ci, b):
        idx_slice = idx_v.at[pl.ds(ci * RPC, RPC)]
        return pltpu.async_copy(cat_hbm.at[idx_slice], rows_v.at[b], gsem[b])

    _gather(0, 0)

    def pair_body(g, carry):
        for b in range(2):
            ci = 2 * g + b
            nci = jnp.minimum(ci + 1, NCHUNK - 1)
            _gather(nci, 1 - b)
            pltpu.make_async_copy(
                cat_hbm.at[idx_v.at[pl.ds(ci * RPC, RPC)]],
                rows_v.at[b], gsem[b]).wait()

            @pl.when(ci >= 2)
            def _():
                pltpu.make_async_copy(
                    out_v.at[b],
                    out_hbm.at[pl.ds(start_w + (ci - 2) * CHUNK, CHUNK)],
                    osem[b]).wait()

            def phrase_body(p, c2):
                base = p * LMAX
                mrow = ci * CHUNK + p
                mb = [mb_v[mrow, pl.ds(16 * l, 16)] for l in range(LMAX)]
                for j in range(C // 16):
                    zb = [rows_v[b, base + l, pl.ds(C + 16 * j, 16)]
                          + bpos[l][j] + mb[l] for l in range(LMAX)]
                    m = zb[0]
                    for l in range(1, LMAX):
                        m = jnp.maximum(m, zb[l])
                    e = [jnp.exp(zb[l] - m) for l in range(LMAX)]
                    s = e[0]
                    for l in range(1, LMAX):
                        s = s + e[l]
                    acc = e[0] * rows_v[b, base, pl.ds(16 * j, 16)]
                    for l in range(1, LMAX):
                        acc = acc + e[l] * rows_v[b, base + l,
                                                  pl.ds(16 * j, 16)]
                    out_v[b, p, pl.ds(16 * j, 16)] = acc / s
                return c2

            lax.fori_loop(0, CHUNK, phrase_body, 0)
            pltpu.async_copy(
                out_v.at[b],
                out_hbm.at[pl.ds(start_w + ci * CHUNK, CHUNK)], osem[b])
        return carry

    lax.fori_loop(0, NCHUNK // 2, pair_body, 0)

    pltpu.make_async_copy(
        cat_hbm.at[idx_v.at[pl.ds((NCHUNK - 1) * RPC, RPC)]],
        rows_v.at[0], gsem[0]).wait()
    for b in range(2):
        ci = NCHUNK - 2 + b
        pltpu.make_async_copy(
            out_v.at[b],
            out_hbm.at[pl.ds(start_w + ci * CHUNK, CHUNK)], osem[b]).wait()


def kernel(h, phrase_mask, phrase_token_idx, W_kv, W_z, B_pos):
    x = h.reshape(B * T, D)
    w_cat = jnp.concatenate([W_kv, W_z], axis=1)
    cat = _project(x, w_cat)
    idx_flat = phrase_token_idx.astype(jnp.int32).reshape(-1)
    mb = jnp.where(phrase_mask, 0.0, _NEG).astype(jnp.float32)
    mb_exp = jnp.broadcast_to(
        mb.reshape(BP, LMAX)[:, :, None], (BP, LMAX, 16)).reshape(BP, LMAX * 16)
    out = _sc_pool(cat, idx_flat, mb_exp,
                   B_pos.astype(jnp.float32).reshape(-1))
    return out.reshape(B, P, C)


# speedup vs baseline: 18.0593x; 1.0778x over previous
import functools

import jax
import jax.numpy as jnp
from jax import lax
from jax.experimental import pallas as pl
from jax.experimental.pallas import tpu as pltpu
from jax.experimental.pallas import tpu_sc as plsc

B, T, D = 4, 8192, 768
P, LMAX, C = 4096, 8, 64
BP = B * P
CAT = 2 * C
NC, NS = 2, 16
NW = NC * NS
PPW = BP // NW
CHUNK = 16
NCHUNK = PPW // CHUNK
RPC = CHUNK * LMAX

_NEG = -1e30


def _mm_body(x_ref, w_ref, o_ref):
    o_ref[...] = jnp.dot(x_ref[...], w_ref[...],
                         preferred_element_type=jnp.float32)


def _project(x, w_cat):
    bm = 1024
    return pl.pallas_call(
        _mm_body,
        grid=(x.shape[0] // bm,),
        in_specs=[pl.BlockSpec((bm, D), lambda i: (i, 0)),
                  pl.BlockSpec((D, CAT), lambda i: (0, 0))],
        out_specs=pl.BlockSpec((bm, CAT), lambda i: (i, 0)),
        out_shape=jax.ShapeDtypeStruct((x.shape[0], CAT), jnp.float32),
    )(x, w_cat)


_mesh = plsc.VectorSubcoreMesh(core_axis_name="c", subcore_axis_name="s")


@functools.partial(
    pl.kernel,
    mesh=_mesh,
    out_type=jax.ShapeDtypeStruct((B, P, C), jnp.float32),
    scratch_types=[
        pltpu.VMEM((PPW * LMAX,), jnp.int32),
        pltpu.VMEM((PPW, LMAX * 16), jnp.float32),
        pltpu.VMEM((LMAX * C,), jnp.float32),
        pltpu.VMEM((2, RPC, CAT), jnp.float32),
        pltpu.VMEM((2, CHUNK, C), jnp.float32),
        pltpu.SemaphoreType.DMA,
        pltpu.SemaphoreType.DMA,
        pltpu.SemaphoreType.DMA,
        pltpu.SemaphoreType.DMA,
    ],
)
def _sc_pool(cat_hbm, idx_hbm, mb_hbm, bpos_hbm, out_hbm,
             idx_v, mb_v, bpos_v, rows_v, out_v, gsem0, gsem1, osem0, osem1):
    wid = lax.axis_index("s") * NC + lax.axis_index("c")
    bb = wid // (NW // B)
    tok_off = bb * T
    start_w = wid * PPW
    pstart_w = start_w - bb * P
    gsem = (gsem0, gsem1)
    osem = (osem0, osem1)

    pltpu.sync_copy(bpos_hbm, bpos_v)
    pltpu.sync_copy(idx_hbm.at[pl.ds(start_w * LMAX, PPW * LMAX)], idx_v)
    pltpu.sync_copy(mb_hbm.at[pl.ds(start_w, PPW)], mb_v)

    def add_off(i, carry):
        sl = pl.ds(16 * i, 16)
        idx_v[sl] = idx_v[sl] + tok_off
        return carry

    lax.fori_loop(0, PPW * LMAX // 16, add_off, 0)

    bpos = [[bpos_v[pl.ds(l * C + 16 * j, 16)] for j in range(C // 16)]
            for l in range(LMAX)]

    def _gather(ci, b):
        idx_slice = idx_v.at[pl.ds(ci * RPC, RPC)]
        return pltpu.async_copy(cat_hbm.at[idx_slice], rows_v.at[b], gsem[b])

    _gather(0, 0)

    def pair_body(g, carry):
        for b in range(2):
            ci = 2 * g + b
            nci = jnp.minimum(ci + 1, NCHUNK - 1)
            _gather(nci, 1 - b)
            pltpu.make_async_copy(
                cat_hbm.at[idx_v.at[pl.ds(ci * RPC, RPC)]],
                rows_v.at[b], gsem[b]).wait()

            @pl.when(ci >= 2)
            def _():
                pltpu.make_async_copy(
                    out_v.at[b],
                    out_hbm.at[bb, pl.ds(pstart_w + (ci - 2) * CHUNK, CHUNK)],
                    osem[b]).wait()

            @plsc.parallel_loop(0, CHUNK, unroll=2)
            def phrase_body(p):
                base = p * LMAX
                mrow = ci * CHUNK + p
                mb = [mb_v[mrow, pl.ds(16 * l, 16)] for l in range(LMAX)]
                for j in range(C // 16):
                    e = [jnp.exp(rows_v[b, base + l, pl.ds(C + 16 * j, 16)]
                                 + bpos[l][j] + mb[l]) for l in range(LMAX)]
                    s = e[0]
                    for l in range(1, LMAX):
                        s = s + e[l]
                    acc = e[0] * rows_v[b, base, pl.ds(16 * j, 16)]
                    for l in range(1, LMAX):
                        acc = acc + e[l] * rows_v[b, base + l,
                                                  pl.ds(16 * j, 16)]
                    out_v[b, p, pl.ds(16 * j, 16)] = acc / s

            pltpu.async_copy(
                out_v.at[b],
                out_hbm.at[bb, pl.ds(pstart_w + ci * CHUNK, CHUNK)], osem[b])
        return carry

    lax.fori_loop(0, NCHUNK // 2, pair_body, 0)

    pltpu.make_async_copy(
        cat_hbm.at[idx_v.at[pl.ds((NCHUNK - 1) * RPC, RPC)]],
        rows_v.at[0], gsem[0]).wait()
    for b in range(2):
        ci = NCHUNK - 2 + b
        pltpu.make_async_copy(
            out_v.at[b],
            out_hbm.at[bb, pl.ds(pstart_w + ci * CHUNK, CHUNK)],
            osem[b]).wait()


def kernel(h, phrase_mask, phrase_token_idx, W_kv, W_z, B_pos):
    x = h.reshape(B * T, D)
    w_cat = jnp.concatenate([W_kv, W_z], axis=1)
    cat = _project(x, w_cat)
    idx_flat = phrase_token_idx.astype(jnp.int32).reshape(-1)
    mb = jnp.where(phrase_mask, 0.0, _NEG).astype(jnp.float32)
    mb_exp = jnp.broadcast_to(
        mb.reshape(BP, LMAX)[:, :, None], (BP, LMAX, 16)).reshape(BP, LMAX * 16)
    return _sc_pool(cat, idx_flat, mb_exp,
                    B_pos.astype(jnp.float32).reshape(-1))


# speedup vs baseline: 19.2010x; 1.0632x over previous
import functools

import jax
import jax.numpy as jnp
from jax import lax
from jax.experimental import pallas as pl
from jax.experimental.pallas import tpu as pltpu
from jax.experimental.pallas import tpu_sc as plsc

B, T, D = 4, 8192, 768
P, LMAX, C = 4096, 8, 64
BP = B * P
CAT = 2 * C
NC, NS = 2, 16
NW = NC * NS
PPW = BP // NW
CHUNK = 16
NCHUNK = PPW // CHUNK
RPC = CHUNK * LMAX

_NEG = -1e30


def _mm_body(x_ref, w_ref, o_ref):
    o_ref[...] = jnp.dot(x_ref[...], w_ref[...],
                         preferred_element_type=jnp.float32)


def _project(x, w_cat):
    bm = 2048
    return pl.pallas_call(
        _mm_body,
        grid=(x.shape[0] // bm,),
        in_specs=[pl.BlockSpec((bm, D), lambda i: (i, 0)),
                  pl.BlockSpec((D, CAT), lambda i: (0, 0))],
        out_specs=pl.BlockSpec((bm, CAT), lambda i: (i, 0)),
        out_shape=jax.ShapeDtypeStruct((x.shape[0], CAT), jnp.float32),
    )(x, w_cat)


_mesh = plsc.VectorSubcoreMesh(core_axis_name="c", subcore_axis_name="s")


@functools.partial(
    pl.kernel,
    mesh=_mesh,
    out_type=jax.ShapeDtypeStruct((B, P, C), jnp.float32),
    scratch_types=[
        pltpu.VMEM((PPW * LMAX,), jnp.int32),
        pltpu.VMEM((PPW, LMAX * 16), jnp.float32),
        pltpu.VMEM((LMAX * C,), jnp.float32),
        pltpu.VMEM((2, RPC, CAT), jnp.float32),
        pltpu.VMEM((2, CHUNK, C), jnp.float32),
        pltpu.SemaphoreType.DMA,
        pltpu.SemaphoreType.DMA,
        pltpu.SemaphoreType.DMA,
        pltpu.SemaphoreType.DMA,
    ],
)
def _sc_pool(cat_hbm, idx_hbm, mb_hbm, bpos_hbm, out_hbm,
             idx_v, mb_v, bpos_v, rows_v, out_v, gsem0, gsem1, osem0, osem1):
    wid = lax.axis_index("s") * NC + lax.axis_index("c")
    bb = wid // (NW // B)
    tok_off = bb * T
    start_w = wid * PPW
    pstart_w = start_w - bb * P
    gsem = (gsem0, gsem1)
    osem = (osem0, osem1)

    pltpu.sync_copy(bpos_hbm, bpos_v)
    pltpu.sync_copy(idx_hbm.at[pl.ds(start_w * LMAX, PPW * LMAX)], idx_v)
    pltpu.sync_copy(mb_hbm.at[pl.ds(start_w, PPW)], mb_v)

    def add_off(i, carry):
        sl = pl.ds(16 * i, 16)
        idx_v[sl] = idx_v[sl] + tok_off
        return carry

    lax.fori_loop(0, PPW * LMAX // 16, add_off, 0)

    bpos = [[bpos_v[pl.ds(l * C + 16 * j, 16)] for j in range(C // 16)]
            for l in range(LMAX)]

    def _gather(ci, b):
        idx_slice = idx_v.at[pl.ds(ci * RPC, RPC)]
        return pltpu.async_copy(cat_hbm.at[idx_slice], rows_v.at[b], gsem[b])

    _gather(0, 0)

    def pair_body(g, carry):
        for b in range(2):
            ci = 2 * g + b
            nci = jnp.minimum(ci + 1, NCHUNK - 1)
            _gather(nci, 1 - b)
            pltpu.make_async_copy(
                cat_hbm.at[idx_v.at[pl.ds(ci * RPC, RPC)]],
                rows_v.at[b], gsem[b]).wait()

            @pl.when(ci >= 2)
            def _():
                pltpu.make_async_copy(
                    out_v.at[b],
                    out_hbm.at[bb, pl.ds(pstart_w + (ci - 2) * CHUNK, CHUNK)],
                    osem[b]).wait()

            @plsc.parallel_loop(0, CHUNK, unroll=4)
            def phrase_body(p):
                base = p * LMAX
                mrow = ci * CHUNK + p
                mb = [mb_v[mrow, pl.ds(16 * l, 16)] for l in range(LMAX)]
                for j in range(C // 16):
                    e = [jnp.exp(rows_v[b, base + l, pl.ds(C + 16 * j, 16)]
                                 + bpos[l][j] + mb[l]) for l in range(LMAX)]
                    s = e[0]
                    for l in range(1, LMAX):
                        s = s + e[l]
                    acc = e[0] * rows_v[b, base, pl.ds(16 * j, 16)]
                    for l in range(1, LMAX):
                        acc = acc + e[l] * rows_v[b, base + l,
                                                  pl.ds(16 * j, 16)]
                    out_v[b, p, pl.ds(16 * j, 16)] = acc / s

            pltpu.async_copy(
                out_v.at[b],
                out_hbm.at[bb, pl.ds(pstart_w + ci * CHUNK, CHUNK)], osem[b])
        return carry

    lax.fori_loop(0, NCHUNK // 2, pair_body, 0)

    pltpu.make_async_copy(
        cat_hbm.at[idx_v.at[pl.ds((NCHUNK - 1) * RPC, RPC)]],
        rows_v.at[0], gsem[0]).wait()
    for b in range(2):
        ci = NCHUNK - 2 + b
        pltpu.make_async_copy(
            out_v.at[b],
            out_hbm.at[bb, pl.ds(pstart_w + ci * CHUNK, CHUNK)],
            osem[b]).wait()


def kernel(h, phrase_mask, phrase_token_idx, W_kv, W_z, B_pos):
    x = h.reshape(B * T, D)
    w_cat = jnp.concatenate([W_kv, W_z], axis=1)
    cat = _project(x, w_cat)
    idx_flat = phrase_token_idx.astype(jnp.int32).reshape(-1)
    mb = jnp.where(phrase_mask, 0.0, _NEG).astype(jnp.float32)
    mb_exp = jnp.broadcast_to(
        mb.reshape(BP, LMAX)[:, :, None], (BP, LMAX, 16)).reshape(BP, LMAX * 16)
    return _sc_pool(cat, idx_flat, mb_exp,
                    B_pos.astype(jnp.float32).reshape(-1))
